# baseline (speedup 1.0000x reference)
import functools
import numpy as np
import jax
import jax.numpy as jnp
from jax import lax
from jax.experimental import pallas as pl
from jax.experimental.pallas import tpu as pltpu
from jax.experimental.pallas import tpu_sc as plsc

_SCALES = (32, 48, 64, 96)
_STRIDE = 16
_H, _W = 96, 128
_NWIN = [((_H - s) // _STRIDE + 1, (_W - s) // _STRIDE + 1) for s in _SCALES]
_KPAD = 80
_NBAND = 6
_BROWS = 4 * _NBAND
_NC, _NS = 2, 16
_NW = _NC * _NS
_B = 512
_BPW = _B // _NW


def _build_weights():
    h = np.arange(_H)
    w = np.arange(_W)
    rmask_band = np.zeros((_BROWS, _KPAD), dtype=np.float64)
    wc = np.zeros((_KPAD, _W), dtype=np.float64)
    off = 0
    for i, s in enumerate(_SCALES):
        ny, nx = _NWIN[i]
        y = np.arange(ny) * _STRIDE
        x = np.arange(nx) * _STRIDE
        Ry = ((h[None, :] >= y[:, None]) & (h[None, :] < y[:, None] + s)).astype(np.float64)
        Cx = ((w[None, :] >= x[:, None]) & (w[None, :] < x[:, None] + s)).astype(np.float64)
        county = Ry.sum(0)
        countx = Cx.sum(0)
        for yy in range(ny):
            for xx in range(nx):
                p = off + yy * nx + xx
                rmask_band[i * _NBAND:(i + 1) * _NBAND, p] = (Ry[yy] / county)[::_STRIDE][:_NBAND]
                wc[p, :] = Cx[xx] / countx
        off += ny * nx
    return rmask_band.astype(np.float32), wc.astype(np.float32)


_RMB_NP, _WC_NP = _build_weights()


def _bands_body(s_ref, rm_ref, wc_ref, o_ref):
    bb = s_ref.shape[0]
    t = (s_ref[...][:, None, :] * rm_ref[...][None, :, :]).reshape(bb * _BROWS, _KPAD)
    r = jnp.dot(t, wc_ref[...], preferred_element_type=jnp.float32)
    o_ref[...] = r.reshape(bb, _BROWS, _W)


_SC_MESH = plsc.VectorSubcoreMesh(core_axis_name="c", subcore_axis_name="s")


@functools.partial(
    pl.kernel,
    mesh=_SC_MESH,
    out_type=jax.ShapeDtypeStruct((_B, 4, _H, _W), jnp.float32),
    scratch_types=[
        pltpu.VMEM((_BROWS, _W), jnp.float32),
        pltpu.VMEM((4, _H, _W), jnp.float32),
    ],
)
def _sc_replicate(bands_hbm, out_hbm, band_v, obuf_v):
    wid = lax.axis_index("s") * _NC + lax.axis_index("c")
    base = wid * _BPW

    def per_b(bi, carry):
        b = base + bi
        pltpu.sync_copy(bands_hbm.at[b], band_v)
        for i in range(4):
            for k in range(_NBAND):
                for c in range(_W // 16):
                    v = band_v[i * _NBAND + k, pl.ds(c * 16, 16)]
                    for r in range(16):
                        obuf_v[i, k * 16 + r, pl.ds(c * 16, 16)] = v
        pltpu.sync_copy(obuf_v, out_hbm.at[b])
        return carry

    lax.fori_loop(0, _BPW, per_b, 0)


def kernel(sim0, sim1, sim2, sim3):
    B = sim0.shape[0]
    parts = [s.reshape(B, -1) for s in (sim0, sim1, sim2, sim3)]
    scat = jnp.concatenate(parts, axis=1)
    scat = jnp.pad(scat, ((0, 0), (0, _KPAD - scat.shape[1])))
    rmb = jnp.asarray(_RMB_NP)
    wc = jnp.asarray(_WC_NP)
    BB = 128
    bands = pl.pallas_call(
        _bands_body,
        grid=(B // BB,),
        in_specs=[
            pl.BlockSpec((BB, _KPAD), lambda i: (i, 0)),
            pl.BlockSpec((_BROWS, _KPAD), lambda i: (0, 0)),
            pl.BlockSpec((_KPAD, _W), lambda i: (0, 0)),
        ],
        out_specs=pl.BlockSpec((BB, _BROWS, _W), lambda i: (i, 0, 0)),
        out_shape=jax.ShapeDtypeStruct((B, _BROWS, _W), jnp.float32),
    )(scat, rmb, wc)
    return _sc_replicate(bands)


# speedup vs baseline: 4.0617x; 4.0617x over previous
import numpy as np
import jax
import jax.numpy as jnp
from jax.experimental import pallas as pl

_SCALES = (32, 48, 64, 96)
_STRIDE = 16
_H, _W = 96, 128
_NWIN = [( (_H - s) // _STRIDE + 1, (_W - s) // _STRIDE + 1) for s in _SCALES]
_NP_TOT = sum(ny * nx for ny, nx in _NWIN)
_KPAD = 80
_ROWS = 4 * _H


def _build_weights():
    h = np.arange(_H)
    w = np.arange(_W)
    rmask = np.zeros((_ROWS, _KPAD), dtype=np.float64)
    wc = np.zeros((_KPAD, _W), dtype=np.float64)
    off = 0
    for i, s in enumerate(_SCALES):
        ny, nx = _NWIN[i]
        y = np.arange(ny) * _STRIDE
        x = np.arange(nx) * _STRIDE
        Ry = ((h[None, :] >= y[:, None]) & (h[None, :] < y[:, None] + s)).astype(np.float64)
        Cx = ((w[None, :] >= x[:, None]) & (w[None, :] < x[:, None] + s)).astype(np.float64)
        county = Ry.sum(0)
        countx = Cx.sum(0)
        for yy in range(ny):
            for xx in range(nx):
                p = off + yy * nx + xx
                rmask[i * _H:(i + 1) * _H, p] = Ry[yy] / county
                wc[p, :] = Cx[xx] / countx
        off += ny * nx
    return rmask.astype(np.float32), wc.astype(np.float32)


_RMASK_NP, _WC_NP = _build_weights()


def _recon_body(s_ref, rm_ref, wc_ref, o_ref):
    bb = s_ref.shape[0]
    t = (s_ref[...][:, None, :] * rm_ref[...][None, :, :]).reshape(bb * _ROWS, _KPAD)
    r = jnp.dot(t, wc_ref[...], preferred_element_type=jnp.float32)
    o_ref[...] = r.reshape(bb, 4, _H, _W)


def kernel(sim0, sim1, sim2, sim3):
    B = sim0.shape[0]
    parts = [s.reshape(B, -1) for s in (sim0, sim1, sim2, sim3)]
    scat = jnp.concatenate(parts, axis=1)
    scat = jnp.pad(scat, ((0, 0), (0, _KPAD - scat.shape[1])))
    rmask = jnp.asarray(_RMASK_NP)
    wc = jnp.asarray(_WC_NP)
    BB = 32
    out = pl.pallas_call(
        _recon_body,
        grid=(B // BB,),
        in_specs=[
            pl.BlockSpec((BB, _KPAD), lambda i: (i, 0)),
            pl.BlockSpec((_ROWS, _KPAD), lambda i: (0, 0)),
            pl.BlockSpec((_KPAD, _W), lambda i: (0, 0)),
        ],
        out_specs=pl.BlockSpec((BB, 4, _H, _W), lambda i: (i, 0, 0, 0)),
        out_shape=jax.ShapeDtypeStruct((B, 4, _H, _W), jnp.float32),
    )(scat, rmask, wc)
    return out
